# Initial kernel scaffold; baseline (speedup 1.0000x reference)
import jax
import jax.numpy as jnp
from jax.experimental import pallas as pl


def kernel(features, edge_index, edge_weights, W1, W2, Wc, gamma1, beta1, alpha1, gamma2, beta2, alpha2):
    raise NotImplementedError("write your pallas kernel here")


# speedup vs baseline: 1.0439x; 1.0439x over previous
import jax
import jax.numpy as jnp
from jax.experimental import pallas as pl
from jax.experimental.pallas import tpu as pltpu

N = 10000
E = 320000
D = 128
H = 128
OUT = 16
EPS = 1e-05
SLOPE = 0.01

_BLK = 1000


def _mm_kernel(x_ref, w_ref, o_ref):
    o_ref[...] = jnp.dot(x_ref[...], w_ref[...],
                         preferred_element_type=jnp.float32)


def _matmul(x, w):
    m, k = x.shape
    _, n = w.shape
    return pl.pallas_call(
        _mm_kernel,
        grid=(m // _BLK,),
        in_specs=[
            pl.BlockSpec((_BLK, k), lambda i: (i, 0)),
            pl.BlockSpec((k, n), lambda i: (0, 0)),
        ],
        out_specs=pl.BlockSpec((_BLK, n), lambda i: (i, 0)),
        out_shape=jax.ShapeDtypeStruct((m, n), jnp.float32),
    )(x, w)


def _leaky(x):
    return jnp.where(x >= 0, x, SLOPE * x)


def _graph_norm(x, gamma, beta, alpha):
    mean = jnp.mean(x, axis=0, keepdims=True)
    h = x - alpha * mean
    var = jnp.mean(h * h, axis=0, keepdims=True)
    return gamma * h / jnp.sqrt(var + EPS) + beta


def kernel(features, edge_index, edge_weights, W1, W2, Wc,
           gamma1, beta1, alpha1, gamma2, beta2, alpha2):
    src = edge_index[0]
    dst = edge_index[1]
    ones = jnp.ones((E,), dtype=jnp.float32)
    deg_out = jax.ops.segment_sum(ones, src, num_segments=N)
    deg_in = jax.ops.segment_sum(ones, dst, num_segments=N)
    s_out = jax.lax.rsqrt(jnp.clip(deg_out, 1.0, None))
    s_in = jax.lax.rsqrt(jnp.clip(deg_in, 1.0, None))

    def layer(x, W, gamma, beta, alpha):
        z = _matmul(x * s_out[:, None], W)
        msg = z[src] * edge_weights[:, None]
        agg = jax.ops.segment_sum(msg, dst, num_segments=N)
        h = agg * s_in[:, None]
        h = _graph_norm(h, gamma, beta, alpha)
        return _leaky(h)

    h1 = layer(features, W1, gamma1, beta1, alpha1)
    r1 = jnp.mean(h1, axis=0, keepdims=True)
    h2 = layer(h1, W2, gamma2, beta2, alpha2)
    r2 = jnp.mean(h2, axis=0, keepdims=True)
    readouts = jnp.hstack((r1, r2))
    return readouts @ Wc.T


# speedup vs baseline: 5.7246x; 5.4836x over previous
import functools

import jax
import jax.numpy as jnp
from jax import lax
from jax.experimental import pallas as pl
from jax.experimental.pallas import tpu as pltpu
from jax.experimental.pallas import tpu_sc as plsc

N = 10000
E = 320000
D = 128
H = 128
OUT = 16
EPS = 1e-05
SLOPE = 0.01

NC = 2
NS = 16
NW = NC * NS
EPW = E // NW
CHUNK = 400
NCHUNK = EPW // CHUNK
GROUPS = CHUNK // 16
ECHUNK = 336
ENCHUNK = 29
ETAIL = EPW - ENCHUNK * ECHUNK
RPT = 624
TAIL = N - NS * RPT

_SC_MESH = plsc.VectorSubcoreMesh(core_axis_name="c", subcore_axis_name="s",
                                  num_cores=NC, num_subcores=NS)
_SC_PARAMS = pltpu.CompilerParams(needs_layout_passes=False)


@functools.partial(
    pl.kernel,
    out_type=jax.ShapeDtypeStruct((NW * 2 * N,), jnp.float32),
    mesh=_SC_MESH,
    compiler_params=_SC_PARAMS,
    scratch_types=[
        pltpu.VMEM((2 * N,), jnp.float32),
        pltpu.VMEM((CHUNK,), jnp.int32),
        pltpu.VMEM((CHUNK,), jnp.int32),
    ],
)
def _deg_kernel(src_hbm, dst_hbm, out_hbm, acc, srcv, dstv):
    c = lax.axis_index("c")
    s = lax.axis_index("s")
    wid = c * NS + s
    zeros16 = jnp.zeros((16,), jnp.float32)

    def zero_body(i, _):
        acc[pl.ds(i * 16, 16)] = zeros16
        return 0

    lax.fori_loop(0, 2 * N // 16, zero_body, 0)

    ones16 = jnp.ones((16,), jnp.float32)
    offN = jnp.full((16,), N, jnp.int32)

    def chunk_body(i, _):
        base = wid * EPW + i * CHUNK
        pltpu.sync_copy(src_hbm.at[pl.ds(base, CHUNK)], srcv)
        pltpu.sync_copy(dst_hbm.at[pl.ds(base, CHUNK)], dstv)

        def group_body(g, _):
            sv = srcv[pl.ds(g * 16, 16)]
            dv = dstv[pl.ds(g * 16, 16)]
            plsc.addupdate_scatter(acc, [sv], ones16)
            plsc.addupdate_scatter(acc, [dv + offN], ones16)
            return 0

        lax.fori_loop(0, GROUPS, group_body, 0)
        return 0

    lax.fori_loop(0, NCHUNK, chunk_body, 0)
    pltpu.sync_copy(acc, out_hbm.at[pl.ds(wid * 2 * N, 2 * N)])


@functools.partial(
    pl.kernel,
    out_type=jax.ShapeDtypeStruct((NC * N, H), jnp.float32),
    mesh=_SC_MESH,
    compiler_params=_SC_PARAMS,
    scratch_types=[
        pltpu.VMEM_SHARED((N, H), jnp.float32),
        pltpu.VMEM((ECHUNK, H), jnp.float32),
        pltpu.VMEM((ECHUNK,), jnp.int32),
        pltpu.VMEM((ECHUNK,), jnp.int32),
        pltpu.VMEM((ECHUNK,), jnp.float32),
    ],
)
def _edge_kernel(z_hbm, src_hbm, dst_hbm, ew_hbm, zinit_hbm, out_hbm,
                 acc, rows, srcv, dstv, eww):
    c = lax.axis_index("c")
    s = lax.axis_index("s")
    wid = c * NS + s

    pltpu.sync_copy(zinit_hbm, acc.at[pl.ds(s * RPT, RPT)])

    @pl.when(s == NS - 1)
    def _():
        pltpu.sync_copy(zinit_hbm.at[pl.ds(0, TAIL)],
                        acc.at[pl.ds(NS * RPT, TAIL)])

    plsc.subcore_barrier()

    def process(base, cn):
        sv = srcv.at[pl.ds(0, cn)]
        dv = dstv.at[pl.ds(0, cn)]
        ev = eww.at[pl.ds(0, cn)]
        rw = rows.at[pl.ds(0, cn)]
        pltpu.sync_copy(src_hbm.at[pl.ds(base, cn)], sv)
        pltpu.sync_copy(dst_hbm.at[pl.ds(base, cn)], dv)
        pltpu.sync_copy(ew_hbm.at[pl.ds(base, cn)], ev)
        pltpu.sync_copy(z_hbm.at[sv], rw)

        def group_body(g, _):
            for j in range(16):
                w = plsc.load_gather(
                    eww, [jnp.full((16,), g * 16 + j, jnp.int32)])
                e = g * 16 + j
                for k in range(H // 16):
                    sl = pl.ds(k * 16, 16)
                    rows[e, sl] = rows[e, sl] * w
            return 0

        lax.fori_loop(0, cn // 16, group_body, 0)
        pltpu.sync_copy(rw, acc.at[dv], add=True)

    def chunk_body(i, _):
        process(wid * EPW + i * ECHUNK, ECHUNK)
        return 0

    lax.fori_loop(0, ENCHUNK, chunk_body, 0)
    process(wid * EPW + ENCHUNK * ECHUNK, ETAIL)
    plsc.subcore_barrier()
    pltpu.sync_copy(acc.at[pl.ds(s * RPT, RPT)],
                    out_hbm.at[pl.ds(c * N + s * RPT, RPT)])

    @pl.when(s == NS - 1)
    def _():
        pltpu.sync_copy(acc.at[pl.ds(NS * RPT, TAIL)],
                        out_hbm.at[pl.ds(c * N + NS * RPT, TAIL)])


_BLK = 1000
_GRID = N // _BLK


def _degsum_body(p_ref, o_ref):
    deg = jnp.sum(p_ref[...], axis=0)
    o_ref[...] = lax.rsqrt(jnp.clip(deg, 1.0, None))


def _deg_scales(parts):
    return pl.pallas_call(
        _degsum_body,
        in_specs=[pl.BlockSpec((NW, 2, N), lambda: (0, 0, 0))],
        out_specs=pl.BlockSpec((2, N), lambda: (0, 0)),
        out_shape=jax.ShapeDtypeStruct((2, N), jnp.float32),
    )(parts)


def _mm_body(x_ref, s_ref, w_ref, o_ref):
    o_ref[...] = jnp.dot(x_ref[...] * s_ref[...], w_ref[...],
                         preferred_element_type=jnp.float32)


def _scaled_matmul(x, s_col, w):
    return pl.pallas_call(
        _mm_body,
        grid=(_GRID,),
        in_specs=[
            pl.BlockSpec((_BLK, D), lambda i: (i, 0)),
            pl.BlockSpec((_BLK, 1), lambda i: (i, 0)),
            pl.BlockSpec((D, H), lambda i: (0, 0)),
        ],
        out_specs=pl.BlockSpec((_BLK, H), lambda i: (i, 0)),
        out_shape=jax.ShapeDtypeStruct((N, H), jnp.float32),
    )(x, s_col, w)


def _stats_body(p_ref, s_ref, h_ref, sums_ref):
    h = (p_ref[0] + p_ref[1]) * s_ref[...]
    h_ref[...] = h

    @pl.when(pl.program_id(0) == 0)
    def _():
        sums_ref[...] = jnp.zeros_like(sums_ref)

    sums_ref[...] += jnp.stack(
        (jnp.sum(h, axis=0), jnp.sum(h * h, axis=0)))


def _combine_stats(parts, s_col):
    return pl.pallas_call(
        _stats_body,
        grid=(_GRID,),
        in_specs=[
            pl.BlockSpec((2, _BLK, H), lambda i: (0, i, 0)),
            pl.BlockSpec((_BLK, 1), lambda i: (i, 0)),
        ],
        out_specs=[
            pl.BlockSpec((_BLK, H), lambda i: (i, 0)),
            pl.BlockSpec((2, H), lambda i: (0, 0)),
        ],
        out_shape=[
            jax.ShapeDtypeStruct((N, H), jnp.float32),
            jax.ShapeDtypeStruct((2, H), jnp.float32),
        ],
    )(parts, s_col)


def _affine(stats_ref, gam_ref, bet_ref, alp_ref):
    mean = stats_ref[0:1] * (1.0 / N)
    e2 = stats_ref[1:2] * (1.0 / N)
    am = alp_ref[...] * mean
    var = e2 - 2.0 * am * mean + am * am
    scale = gam_ref[...] * lax.rsqrt(var + EPS)
    shift = bet_ref[...] - scale * am
    return scale, shift


def _leaky(x):
    return jnp.where(x >= 0, x, SLOPE * x)


def _norm_mm_body(h_ref, stats_ref, gam_ref, bet_ref, alp_ref, s_ref, w_ref,
                  z_ref, rsum_ref):
    scale, shift = _affine(stats_ref, gam_ref, bet_ref, alp_ref)
    h = _leaky(h_ref[...] * scale + shift)

    @pl.when(pl.program_id(0) == 0)
    def _():
        rsum_ref[...] = jnp.zeros_like(rsum_ref)

    rsum_ref[...] += jnp.sum(h, axis=0, keepdims=True)
    z_ref[...] = jnp.dot(h * s_ref[...], w_ref[...],
                         preferred_element_type=jnp.float32)


def _norm_matmul(h_pre, stats, gam, bet, alp, s_col, w):
    return pl.pallas_call(
        _norm_mm_body,
        grid=(_GRID,),
        in_specs=[
            pl.BlockSpec((_BLK, H), lambda i: (i, 0)),
            pl.BlockSpec((2, H), lambda i: (0, 0)),
            pl.BlockSpec((1, H), lambda i: (0, 0)),
            pl.BlockSpec((1, H), lambda i: (0, 0)),
            pl.BlockSpec((1, H), lambda i: (0, 0)),
            pl.BlockSpec((_BLK, 1), lambda i: (i, 0)),
            pl.BlockSpec((H, H), lambda i: (0, 0)),
        ],
        out_specs=[
            pl.BlockSpec((_BLK, H), lambda i: (i, 0)),
            pl.BlockSpec((1, H), lambda i: (0, 0)),
        ],
        out_shape=[
            jax.ShapeDtypeStruct((N, H), jnp.float32),
            jax.ShapeDtypeStruct((1, H), jnp.float32),
        ],
    )(h_pre, stats, gam, bet, alp, s_col, w)


def _final_body(h_ref, stats_ref, gam_ref, bet_ref, alp_ref, r1_ref, wc_ref,
                o_ref, acc):
    scale, shift = _affine(stats_ref, gam_ref, bet_ref, alp_ref)
    h = _leaky(h_ref[...] * scale + shift)

    @pl.when(pl.program_id(0) == 0)
    def _():
        acc[...] = jnp.zeros_like(acc)

    acc[...] += jnp.sum(h, axis=0, keepdims=True)

    @pl.when(pl.program_id(0) == _GRID - 1)
    def _():
        r = jnp.concatenate((r1_ref[...], acc[...]), axis=1) * (1.0 / N)
        o_ref[...] = lax.dot_general(
            r, wc_ref[...], (((1,), (1,)), ((), ())),
            preferred_element_type=jnp.float32)


def _final(h_pre, stats, gam, bet, alp, r1sum, wc):
    return pl.pallas_call(
        _final_body,
        grid=(_GRID,),
        in_specs=[
            pl.BlockSpec((_BLK, H), lambda i: (i, 0)),
            pl.BlockSpec((2, H), lambda i: (0, 0)),
            pl.BlockSpec((1, H), lambda i: (0, 0)),
            pl.BlockSpec((1, H), lambda i: (0, 0)),
            pl.BlockSpec((1, H), lambda i: (0, 0)),
            pl.BlockSpec((1, H), lambda i: (0, 0)),
            pl.BlockSpec((OUT, 2 * H), lambda i: (0, 0)),
        ],
        out_specs=pl.BlockSpec((1, OUT), lambda i: (0, 0)),
        out_shape=jax.ShapeDtypeStruct((1, OUT), jnp.float32),
        scratch_shapes=[pltpu.VMEM((1, H), jnp.float32)],
    )(h_pre, stats, gam, bet, alp, r1sum, wc)


def kernel(features, edge_index, edge_weights, W1, W2, Wc,
           gamma1, beta1, alpha1, gamma2, beta2, alpha2):
    src = edge_index[0]
    dst = edge_index[1]
    zinit = jnp.zeros((RPT, H), jnp.float32)

    degp = _deg_kernel(src, dst).reshape(NW, 2, N)
    sres = _deg_scales(degp)
    s_out = sres[0].reshape(N, 1)
    s_in = sres[1].reshape(N, 1)

    g1 = gamma1.reshape(1, H)
    b1 = beta1.reshape(1, H)
    a1 = alpha1.reshape(1, H)
    g2 = gamma2.reshape(1, H)
    b2 = beta2.reshape(1, H)
    a2 = alpha2.reshape(1, H)

    z1 = _scaled_matmul(features, s_out, W1)
    parts1 = _edge_kernel(z1, src, dst, edge_weights, zinit).reshape(NC, N, H)
    h1_pre, stats1 = _combine_stats(parts1, s_in)

    z2, r1sum = _norm_matmul(h1_pre, stats1, g1, b1, a1, s_out, W2)
    parts2 = _edge_kernel(z2, src, dst, edge_weights, zinit).reshape(NC, N, H)
    h2_pre, stats2 = _combine_stats(parts2, s_in)

    return _final(h2_pre, stats2, g2, b2, a2, r1sum, Wc)
